# baseline (speedup 1.0000x reference)
import functools

import jax
import jax.numpy as jnp
from jax.experimental import pallas as pl
from jax.experimental.pallas import tpu as pltpu

_TOKEN_TILE = 512


def _mlp_kernel(x_ref, w1_ref, b1_ref, w2_ref, b2_ref,
                logits_ref, probs_ref, acts_ref):
    x = x_ref[...]
    hidden = jnp.maximum(
        jnp.dot(x, w1_ref[...], preferred_element_type=jnp.float32)
        + b1_ref[...], 0.0)
    logits = (jnp.dot(hidden, w2_ref[...], preferred_element_type=jnp.float32)
              + b2_ref[...])
    probs = 0.5 * jnp.tanh(0.5 * logits) + 0.5
    logits_ref[...] = logits
    probs_ref[...] = probs
    acts_ref[...] = (logits > 0.0).astype(jnp.float32)


@jax.jit
def kernel(neural_repr, W1, b1, W2, b2):
    tokens, embed = neural_repr.shape
    hidden = W1.shape[1]
    num_symbols = W2.shape[1]
    tile = min(_TOKEN_TILE, tokens)
    grid = (tokens // tile,)

    out_shape = [
        jax.ShapeDtypeStruct((tokens, num_symbols), jnp.float32)
        for _ in range(3)
    ]
    out_spec = pl.BlockSpec((tile, num_symbols), lambda i: (i, 0))

    logits, probs, acts = pl.pallas_call(
        _mlp_kernel,
        grid=grid,
        in_specs=[
            pl.BlockSpec((tile, embed), lambda i: (i, 0)),
            pl.BlockSpec((embed, hidden), lambda i: (0, 0)),
            pl.BlockSpec((1, hidden), lambda i: (0, 0)),
            pl.BlockSpec((hidden, num_symbols), lambda i: (0, 0)),
            pl.BlockSpec((1, num_symbols), lambda i: (0, 0)),
        ],
        out_specs=[out_spec, out_spec, out_spec],
        out_shape=out_shape,
        compiler_params=pltpu.CompilerParams(
            dimension_semantics=("arbitrary",),
        ),
    )(neural_repr, W1, b1.reshape(1, hidden), W2, b2.reshape(1, num_symbols))
    return (logits, probs, acts)


# speedup vs baseline: 1.0235x; 1.0235x over previous
import functools

import jax
import jax.numpy as jnp
from jax.experimental import pallas as pl
from jax.experimental.pallas import tpu as pltpu

_TOKEN_TILE = 2048


def _mlp_kernel(x_ref, w1_ref, b1_ref, w2_ref, b2_ref,
                logits_ref, probs_ref, acts_ref):
    x = x_ref[...]
    hidden = jnp.maximum(
        jnp.dot(x, w1_ref[...], preferred_element_type=jnp.float32)
        + b1_ref[...], 0.0)
    logits = (jnp.dot(hidden, w2_ref[...], preferred_element_type=jnp.float32)
              + b2_ref[...])
    probs = 0.5 * jnp.tanh(0.5 * logits) + 0.5
    logits_ref[...] = logits
    probs_ref[...] = probs
    acts_ref[...] = (logits > 0.0).astype(jnp.float32)


@jax.jit
def kernel(neural_repr, W1, b1, W2, b2):
    tokens, embed = neural_repr.shape
    hidden = W1.shape[1]
    num_symbols = W2.shape[1]
    tile = min(_TOKEN_TILE, tokens)
    grid = (tokens // tile,)

    out_shape = [
        jax.ShapeDtypeStruct((tokens, num_symbols), jnp.float32)
        for _ in range(3)
    ]
    out_spec = pl.BlockSpec((tile, num_symbols), lambda i: (i, 0))

    logits, probs, acts = pl.pallas_call(
        _mlp_kernel,
        grid=grid,
        in_specs=[
            pl.BlockSpec((tile, embed), lambda i: (i, 0)),
            pl.BlockSpec((embed, hidden), lambda i: (0, 0)),
            pl.BlockSpec((1, hidden), lambda i: (0, 0)),
            pl.BlockSpec((hidden, num_symbols), lambda i: (0, 0)),
            pl.BlockSpec((1, num_symbols), lambda i: (0, 0)),
        ],
        out_specs=[out_spec, out_spec, out_spec],
        out_shape=out_shape,
        compiler_params=pltpu.CompilerParams(
            dimension_semantics=("arbitrary",),
        ),
    )(neural_repr, W1, b1.reshape(1, hidden), W2, b2.reshape(1, num_symbols))
    return (logits, probs, acts)


# speedup vs baseline: 1.0293x; 1.0056x over previous
import functools

import jax
import jax.numpy as jnp
from jax.experimental import pallas as pl
from jax.experimental.pallas import tpu as pltpu

_TOKEN_TILE = 1024


def _mlp_kernel(x_ref, w1_ref, b1_ref, w2_ref, b2_ref,
                logits_ref, probs_ref, acts_ref):
    x = x_ref[...]
    hidden = jnp.maximum(
        jnp.dot(x, w1_ref[...], preferred_element_type=jnp.float32)
        + b1_ref[...], 0.0)
    logits = (jnp.dot(hidden, w2_ref[...], preferred_element_type=jnp.float32)
              + b2_ref[...])
    probs = 0.5 * jnp.tanh(0.5 * logits) + 0.5
    logits_ref[...] = logits
    probs_ref[...] = probs
    acts_ref[...] = (logits > 0.0).astype(jnp.float32)


@jax.jit
def kernel(neural_repr, W1, b1, W2, b2):
    tokens, embed = neural_repr.shape
    hidden = W1.shape[1]
    num_symbols = W2.shape[1]
    tile = min(_TOKEN_TILE, tokens)
    grid = (tokens // tile,)

    out_shape = [
        jax.ShapeDtypeStruct((tokens, num_symbols), jnp.float32)
        for _ in range(3)
    ]
    out_spec = pl.BlockSpec((tile, num_symbols), lambda i: (i, 0))

    logits, probs, acts = pl.pallas_call(
        _mlp_kernel,
        grid=grid,
        in_specs=[
            pl.BlockSpec((tile, embed), lambda i: (i, 0)),
            pl.BlockSpec((embed, hidden), lambda i: (0, 0)),
            pl.BlockSpec((1, hidden), lambda i: (0, 0)),
            pl.BlockSpec((hidden, num_symbols), lambda i: (0, 0)),
            pl.BlockSpec((1, num_symbols), lambda i: (0, 0)),
        ],
        out_specs=[out_spec, out_spec, out_spec],
        out_shape=out_shape,
        compiler_params=pltpu.CompilerParams(
            dimension_semantics=("parallel",),
        ),
    )(neural_repr, W1, b1.reshape(1, hidden), W2, b2.reshape(1, num_symbols))
    return (logits, probs, acts)


# speedup vs baseline: 1.0297x; 1.0004x over previous
import jax
import jax.numpy as jnp
from jax.experimental import pallas as pl
from jax.experimental.pallas import tpu as pltpu

_TOKEN_TILE = 1024


def _mlp_kernel(x_ref, w1_ref, b1_ref, w2_ref, b2_ref,
                logits_ref, probs_ref, acts_ref):
    x = x_ref[...]
    hidden = jnp.maximum(
        jnp.dot(x, w1_ref[...], preferred_element_type=jnp.float32)
        + b1_ref[...], 0.0)
    logits = (jnp.dot(hidden, w2_ref[...], preferred_element_type=jnp.float32)
              + b2_ref[...])
    probs = jax.nn.sigmoid(logits)
    logits_ref[...] = logits
    probs_ref[...] = probs
    acts_ref[...] = (probs > 0.5).astype(jnp.float32)


@jax.jit
def kernel(neural_repr, W1, b1, W2, b2):
    tokens, embed = neural_repr.shape
    hidden = W1.shape[1]
    num_symbols = W2.shape[1]
    tile = min(_TOKEN_TILE, tokens)
    grid = (tokens // tile,)

    out_shape = [
        jax.ShapeDtypeStruct((tokens, num_symbols), jnp.float32)
        for _ in range(3)
    ]
    out_spec = pl.BlockSpec((tile, num_symbols), lambda i: (i, 0))

    logits, probs, acts = pl.pallas_call(
        _mlp_kernel,
        grid=grid,
        in_specs=[
            pl.BlockSpec((tile, embed), lambda i: (i, 0)),
            pl.BlockSpec((embed, hidden), lambda i: (0, 0)),
            pl.BlockSpec((1, hidden), lambda i: (0, 0)),
            pl.BlockSpec((hidden, num_symbols), lambda i: (0, 0)),
            pl.BlockSpec((1, num_symbols), lambda i: (0, 0)),
        ],
        out_specs=[out_spec, out_spec, out_spec],
        out_shape=out_shape,
        compiler_params=pltpu.CompilerParams(
            dimension_semantics=("arbitrary",),
        ),
    )(neural_repr, W1, b1.reshape(1, hidden), W2, b2.reshape(1, num_symbols))
    return (logits, probs, acts)


# speedup vs baseline: 1.0305x; 1.0008x over previous
import jax
import jax.numpy as jnp
from jax.experimental import pallas as pl
from jax.experimental.pallas import tpu as pltpu

_TOKEN_TILE = 1024
_N_BUF = 3


def _make_kernel(grid_len, tile):
    def body(x_ref, w1_ref, b1_ref, w2_ref, b2_ref,
             logits_hbm, probs_hbm, acts_hbm,
             l_scr, p_scr, a_scr, sems):
        i = pl.program_id(0)
        slot = jax.lax.rem(i, _N_BUF)

        def copies(j, s):
            base = j * tile
            return [
                pltpu.make_async_copy(
                    l_scr.at[s], logits_hbm.at[pl.ds(base, tile), :],
                    sems.at[s, 0]),
                pltpu.make_async_copy(
                    p_scr.at[s], probs_hbm.at[pl.ds(base, tile), :],
                    sems.at[s, 1]),
                pltpu.make_async_copy(
                    a_scr.at[s], acts_hbm.at[pl.ds(base, tile), :],
                    sems.at[s, 2]),
            ]

        @pl.when(i >= _N_BUF)
        def _():
            for c in copies(i - _N_BUF, slot):
                c.wait()

        x = x_ref[...]
        hidden = jnp.maximum(
            jnp.dot(x, w1_ref[...], preferred_element_type=jnp.float32)
            + b1_ref[...], 0.0)
        logits = (jnp.dot(hidden, w2_ref[...],
                          preferred_element_type=jnp.float32) + b2_ref[...])
        probs = jax.nn.sigmoid(logits)
        l_scr[slot] = logits
        p_scr[slot] = probs
        a_scr[slot] = (probs > 0.5).astype(jnp.float32)
        for c in copies(i, slot):
            c.start()

        @pl.when(i == grid_len - 1)
        def _():
            for k in range(_N_BUF):
                j = grid_len - _N_BUF + k
                for c in copies(j, jax.lax.rem(jnp.int32(j), _N_BUF)):
                    c.wait()

    return body


@jax.jit
def kernel(neural_repr, W1, b1, W2, b2):
    tokens, embed = neural_repr.shape
    hidden = W1.shape[1]
    num_symbols = W2.shape[1]
    tile = min(_TOKEN_TILE, tokens)
    grid_len = tokens // tile

    out_shape = [
        jax.ShapeDtypeStruct((tokens, num_symbols), jnp.float32)
        for _ in range(3)
    ]
    any_spec = pl.BlockSpec(memory_space=pl.ANY)

    logits, probs, acts = pl.pallas_call(
        _make_kernel(grid_len, tile),
        grid=(grid_len,),
        in_specs=[
            pl.BlockSpec((tile, embed), lambda i: (i, 0)),
            pl.BlockSpec((embed, hidden), lambda i: (0, 0)),
            pl.BlockSpec((1, hidden), lambda i: (0, 0)),
            pl.BlockSpec((hidden, num_symbols), lambda i: (0, 0)),
            pl.BlockSpec((1, num_symbols), lambda i: (0, 0)),
        ],
        out_specs=[any_spec, any_spec, any_spec],
        out_shape=out_shape,
        scratch_shapes=[
            pltpu.VMEM((_N_BUF, tile, num_symbols), jnp.float32),
            pltpu.VMEM((_N_BUF, tile, num_symbols), jnp.float32),
            pltpu.VMEM((_N_BUF, tile, num_symbols), jnp.float32),
            pltpu.SemaphoreType.DMA((_N_BUF, 3)),
        ],
        compiler_params=pltpu.CompilerParams(
            dimension_semantics=("arbitrary",),
        ),
    )(neural_repr, W1, b1.reshape(1, hidden), W2, b2.reshape(1, num_symbols))
    return (logits, probs, acts)
